# baseline (speedup 1.0000x reference)
import functools

import jax
import jax.numpy as jnp
from jax import lax
from jax.experimental import pallas as pl
from jax.experimental.pallas import tpu as pltpu
from jax.experimental.pallas import tpu_sc as plsc

NC, NS = 2, 16
NW = NC * NS
B = 16384
D = 64
BPW = B // NW
CH = 128
NCH = BPW // CH
LANES = 16


def _body(user_hbm, ii_hbm, ij_hbm, uw_hbm, iw_hbm, out_i_hbm, out_j_hbm,
          uidx, iidx, jidx, urows, irows, jrows, ti, tj, oi, oj, sem):
    wid = lax.axis_index("s") * NC + lax.axis_index("c")

    pltpu.sync_copy(user_hbm.at[wid], uidx)
    pltpu.sync_copy(ii_hbm.at[wid], iidx)
    pltpu.sync_copy(ij_hbm.at[wid], jidx)

    copies = []
    for c in range(NCH):
        copies.append(pltpu.async_copy(
            uw_hbm.at[uidx.at[c]], urows.at[pl.ds(c * CH, CH)], sem))
        copies.append(pltpu.async_copy(
            iw_hbm.at[iidx.at[c]], irows.at[pl.ds(c * CH, CH)], sem))
        copies.append(pltpu.async_copy(
            iw_hbm.at[jidx.at[c]], jrows.at[pl.ds(c * CH, CH)], sem))
    for cp in copies:
        cp.wait()

    iota = lax.iota(jnp.int32, LANES)

    def group(g, carry):
        base_r = g * LANES
        for m in range(LANES):
            r = base_r + m
            acc_i = urows[r, pl.ds(0, LANES)] * irows[r, pl.ds(0, LANES)]
            acc_j = urows[r, pl.ds(0, LANES)] * jrows[r, pl.ds(0, LANES)]
            for k in range(1, D // LANES):
                u = urows[r, pl.ds(k * LANES, LANES)]
                acc_i = acc_i + u * irows[r, pl.ds(k * LANES, LANES)]
                acc_j = acc_j + u * jrows[r, pl.ds(k * LANES, LANES)]
            tidx = iota * LANES + m
            plsc.store_scatter(ti, [tidx], acc_i)
            plsc.store_scatter(tj, [tidx], acc_j)
        si = ti[pl.ds(0, LANES)]
        sj = tj[pl.ds(0, LANES)]
        for m in range(1, LANES):
            si = si + ti[pl.ds(m * LANES, LANES)]
            sj = sj + tj[pl.ds(m * LANES, LANES)]
        oi[pl.ds(base_r, LANES)] = si
        oj[pl.ds(base_r, LANES)] = sj
        return carry

    lax.fori_loop(0, BPW // LANES, group, 0)

    pltpu.sync_copy(oi, out_i_hbm.at[pl.ds(wid * BPW, BPW)])
    pltpu.sync_copy(oj, out_j_hbm.at[pl.ds(wid * BPW, BPW)])


@jax.jit
def _sc_bpr(user3, ii3, ij3, uw, iw):
    f32 = jnp.float32
    call = pl.kernel(
        _body,
        out_type=(jax.ShapeDtypeStruct((B,), f32),
                  jax.ShapeDtypeStruct((B,), f32)),
        mesh=plsc.VectorSubcoreMesh(
            core_axis_name="c", subcore_axis_name="s",
            num_cores=NC, num_subcores=NS),
        scratch_types=[
            pltpu.VMEM((NCH, CH), jnp.int32),
            pltpu.VMEM((NCH, CH), jnp.int32),
            pltpu.VMEM((NCH, CH), jnp.int32),
            pltpu.VMEM((BPW, D), f32),
            pltpu.VMEM((BPW, D), f32),
            pltpu.VMEM((BPW, D), f32),
            pltpu.VMEM((LANES * LANES,), f32),
            pltpu.VMEM((LANES * LANES,), f32),
            pltpu.VMEM((BPW,), f32),
            pltpu.VMEM((BPW,), f32),
            pltpu.SemaphoreType.DMA,
        ],
        compiler_params=pltpu.CompilerParams(
            needs_layout_passes=False, use_tc_tiling_on_sc=False),
    )
    return call(user3, ii3, ij3, uw, iw)


def kernel(user, item_i, item_j, embed_user_w, embed_item_w):
    user3 = user.astype(jnp.int32).reshape(NW, NCH, CH)
    ii3 = item_i.astype(jnp.int32).reshape(NW, NCH, CH)
    ij3 = item_j.astype(jnp.int32).reshape(NW, NCH, CH)
    return _sc_bpr(user3, ii3, ij3, embed_user_w, embed_item_w)
